# Initial kernel scaffold; baseline (speedup 1.0000x reference)
import jax
import jax.numpy as jnp
from jax.experimental import pallas as pl


def kernel(x, edge_index, new_x, w_attn, W_rel, b_rel, W_root):
    raise NotImplementedError("write your pallas kernel here")


# speedup vs baseline: 61.5070x; 61.5070x over previous
import functools

import jax
import jax.numpy as jnp
from jax import lax
from jax.experimental import pallas as pl
from jax.experimental.pallas import tpu as pltpu
from jax.experimental.pallas import tpu_sc as plsc

N = 10000
CH = 128
E = 320000
MIN_SCORE = 0.7
TOL = 1e-7

NC = 2
NS = 16
L = 16
NW = NC * NS
EPW = E // NW
VPW = EPW // L
ZR = 16
NCHUNK = N // ZR


def _score_body(x_ref, wa_ref, s_ref):
    x = x_ref[...]
    wa = wa_ref[...]
    score = jnp.sum(x * wa, axis=1, keepdims=True)
    m = jnp.max(score)
    e = jnp.exp(score - m)
    sm = e / jnp.sum(e)
    thr = jnp.minimum(jnp.max(sm) - TOL, MIN_SCORE)
    s_ref[...] = jnp.where(sm > thr, sm, 0.0)


def _scores(x, w_attn):
    return pl.pallas_call(
        _score_body,
        out_shape=jax.ShapeDtypeStruct((N, 1), jnp.float32),
    )(x, w_attn.reshape(1, CH))


def _sc_body(x_hbm, row_hbm, col_hbm, s_hbm, out_hbm,
             row_v, col_v, s_v, rows_v, zbuf_v, sem, agg_sh):
    cid = lax.axis_index("c")
    sid = lax.axis_index("s")
    wid = sid * NC + cid
    ebase = wid * EPW
    pltpu.sync_copy(row_hbm.at[pl.ds(ebase, EPW)], row_v)
    pltpu.sync_copy(col_hbm.at[pl.ds(ebase, EPW)], col_v)
    pltpu.sync_copy(s_hbm, s_v)

    zero = jnp.zeros((L,), jnp.float32)
    for r in range(ZR):
        for k in range(CH // L):
            zbuf_v[r, pl.ds(k * L, L)] = zero

    def zchunk(t, carry):
        j = t * NS + sid

        @pl.when(j < NCHUNK)
        def _():
            pltpu.sync_copy(zbuf_v, agg_sh.at[pl.ds(j * ZR, ZR)])

        return carry

    lax.fori_loop(0, pl.cdiv(NCHUNK, NS), zchunk, 0)
    plsc.subcore_barrier()

    lanes = lax.iota(jnp.int32, L)

    def step(i, carry):
        r = row_v[pl.ds(i * L, L)]
        w = plsc.load_gather(s_v, [r])

        @pl.when(jnp.sum(w) > 0.0)
        def _active():
            c = col_v[pl.ds(i * L, L)]
            pltpu.async_copy(x_hbm.at[r], rows_v, sem).wait()
            for lr in range(L):
                wr = jnp.sum(jnp.where(lanes == lr, w, 0.0))
                for k in range(CH // L):
                    sl = pl.ds(k * L, L)
                    rows_v[lr, sl] = rows_v[lr, sl] * wr
            pltpu.sync_copy(rows_v, agg_sh.at[c], add=True)

        return carry

    lax.fori_loop(0, VPW, step, 0)

    plsc.subcore_barrier()

    def ochunk(t, carry):
        j = t * NS + sid

        @pl.when(j < NCHUNK)
        def _():
            off = j * ZR
            pltpu.sync_copy(agg_sh.at[pl.ds(off, ZR)], zbuf_v)
            pltpu.sync_copy(zbuf_v, out_hbm.at[cid, pl.ds(off, ZR)])

        return carry

    lax.fori_loop(0, pl.cdiv(NCHUNK, NS), ochunk, 0)


_sc_call = pl.kernel(
    _sc_body,
    out_type=jax.ShapeDtypeStruct((NC, N, CH), jnp.float32),
    mesh=plsc.VectorSubcoreMesh(
        core_axis_name="c", subcore_axis_name="s",
        num_cores=NC, num_subcores=NS),
    scratch_types=[
        pltpu.VMEM((EPW,), jnp.int32),
        pltpu.VMEM((EPW,), jnp.int32),
        pltpu.VMEM((N,), jnp.float32),
        pltpu.VMEM((L, CH), jnp.float32),
        pltpu.VMEM((ZR, CH), jnp.float32),
        pltpu.SemaphoreType.DMA,
        pltpu.VMEM_SHARED((N, CH), jnp.float32),
    ],
    compiler_params=pltpu.CompilerParams(needs_layout_passes=False),
)


BLK = 1000


def _out_body(s_ref, x_ref, nx_ref, agg_ref, wrel_ref, brel_ref,
              wroot_ref, o_ref):
    s = s_ref[...]
    agg = agg_ref[0] + agg_ref[1]
    xs = x_ref[...] * s
    out = (jnp.dot(agg, wrel_ref[...].T, preferred_element_type=jnp.float32,
                   precision=lax.Precision.HIGHEST)
           + brel_ref[...]
           + jnp.dot(xs, wroot_ref[...].T, preferred_element_type=jnp.float32,
                     precision=lax.Precision.HIGHEST))
    out = jnp.where(out >= 0, out, 0.01 * out)
    o_ref[:, :CH] = nx_ref[...]
    o_ref[:, CH:] = jnp.where(s > 0, out, 0.0)


def _finalize(s, x, new_x, agg2, W_rel, b_rel, W_root):
    return pl.pallas_call(
        _out_body,
        grid=(N // BLK,),
        in_specs=[
            pl.BlockSpec((BLK, 1), lambda i: (i, 0)),
            pl.BlockSpec((BLK, CH), lambda i: (i, 0)),
            pl.BlockSpec((BLK, CH), lambda i: (i, 0)),
            pl.BlockSpec((NC, BLK, CH), lambda i: (0, i, 0)),
            pl.BlockSpec((CH, CH), lambda i: (0, 0)),
            pl.BlockSpec((1, CH), lambda i: (0, 0)),
            pl.BlockSpec((CH, CH), lambda i: (0, 0)),
        ],
        out_specs=pl.BlockSpec((BLK, 2 * CH), lambda i: (i, 0)),
        out_shape=jax.ShapeDtypeStruct((N, 2 * CH), jnp.float32),
    )(s, x, new_x, agg2, W_rel, b_rel.reshape(1, CH), W_root)


@jax.jit
def kernel(x, edge_index, new_x, w_attn, W_rel, b_rel, W_root):
    s = _scores(x, w_attn)
    agg2 = _sc_call(x, edge_index[0], edge_index[1], s.reshape(N))
    return _finalize(s, x, new_x, agg2, W_rel, b_rel, W_root)


# speedup vs baseline: 73.4370x; 1.1940x over previous
import functools

import jax
import jax.numpy as jnp
from jax import lax
from jax.experimental import pallas as pl
from jax.experimental.pallas import tpu as pltpu
from jax.experimental.pallas import tpu_sc as plsc

N = 10000
CH = 128
E = 320000
MIN_SCORE = 0.7
TOL = 1e-7

NC = 2
NS = 16
L = 16
NW = NC * NS
EPW = E // NW
VPW = EPW // L
ZR = 16
NCHUNK = N // ZR


def _score_body(x_ref, wa_ref, s_ref):
    x = x_ref[...]
    wa = wa_ref[...]
    score = jnp.sum(x * wa, axis=1, keepdims=True)
    m = jnp.max(score)
    e = jnp.exp(score - m)
    sm = e / jnp.sum(e)
    thr = jnp.minimum(jnp.max(sm) - TOL, MIN_SCORE)
    s_ref[...] = jnp.where(sm > thr, sm, 0.0)


def _scores(x, w_attn):
    return pl.pallas_call(
        _score_body,
        out_shape=jax.ShapeDtypeStruct((N, 1), jnp.float32),
    )(x, w_attn.reshape(1, CH))


UNROLL = 5
RPS = 624


def _sc_body(x_hbm, row_hbm, col_hbm, s_hbm, zero_hbm, out_hbm,
             row_v, col_v, s_v, rows_v, sem, zsem, agg_sh):
    cid = lax.axis_index("c")
    sid = lax.axis_index("s")
    wid = sid * NC + cid
    ebase = wid * EPW

    rbase = sid * RPS
    zcp = []
    for t in range(RPS // ZR):
        off = rbase + t * ZR
        zcp.append(pltpu.async_copy(
            zero_hbm.at[pl.ds(off, ZR)], agg_sh.at[pl.ds(off, ZR)], zsem))

    @pl.when(sid == NS - 1)
    def _ztail():
        pltpu.async_copy(
            zero_hbm.at[pl.ds(NS * RPS, ZR)],
            agg_sh.at[pl.ds(NS * RPS, ZR)], zsem).wait()

    pltpu.sync_copy(row_hbm.at[pl.ds(ebase, EPW)], row_v)
    pltpu.sync_copy(col_hbm.at[pl.ds(ebase, EPW)], col_v)
    pltpu.sync_copy(s_hbm, s_v)
    for c in zcp:
        c.wait()
    plsc.subcore_barrier()

    lanes = lax.iota(jnp.int32, L)

    def _do_chunk(base, w):
        r = row_v[pl.ds(base, L)]
        c = col_v[pl.ds(base, L)]
        pltpu.async_copy(x_hbm.at[r], rows_v, sem).wait()
        for lr in range(L):
            wr = jnp.sum(jnp.where(lanes == lr, w, 0.0))
            for k in range(CH // L):
                sl = pl.ds(k * L, L)
                rows_v[lr, sl] = rows_v[lr, sl] * wr
        pltpu.sync_copy(rows_v, agg_sh.at[c], add=True)

    def step(i, carry):
        base = i * (L * UNROLL)
        ws = [plsc.load_gather(s_v, [row_v[pl.ds(base + u * L, L)]])
              for u in range(UNROLL)]
        tot = ws[0]
        for u in range(1, UNROLL):
            tot = tot + ws[u]

        @pl.when(jnp.sum(tot) > 0.0)
        def _active():
            for u in range(UNROLL):
                @pl.when(jnp.sum(ws[u]) > 0.0)
                def _chunk(u=u):
                    _do_chunk(base + u * L, ws[u])

        return carry

    lax.fori_loop(0, VPW // UNROLL, step, 0)

    plsc.subcore_barrier()

    ocp = []
    for t in range(RPS // ZR):
        off = rbase + t * ZR
        ocp.append(pltpu.async_copy(
            agg_sh.at[pl.ds(off, ZR)], out_hbm.at[cid, pl.ds(off, ZR)], zsem))

    @pl.when(sid == NS - 1)
    def _otail():
        pltpu.async_copy(
            agg_sh.at[pl.ds(NS * RPS, ZR)],
            out_hbm.at[cid, pl.ds(NS * RPS, ZR)], zsem).wait()

    for c in ocp:
        c.wait()


_sc_call = pl.kernel(
    _sc_body,
    out_type=jax.ShapeDtypeStruct((NC, N, CH), jnp.float32),
    mesh=plsc.VectorSubcoreMesh(
        core_axis_name="c", subcore_axis_name="s",
        num_cores=NC, num_subcores=NS),
    scratch_types=[
        pltpu.VMEM((EPW,), jnp.int32),
        pltpu.VMEM((EPW,), jnp.int32),
        pltpu.VMEM((N,), jnp.float32),
        pltpu.VMEM((L, CH), jnp.float32),
        pltpu.SemaphoreType.DMA,
        pltpu.SemaphoreType.DMA,
        pltpu.VMEM_SHARED((N, CH), jnp.float32),
    ],
    compiler_params=pltpu.CompilerParams(needs_layout_passes=False),
)


BLK = 1000


def _out_body(s_ref, x_ref, nx_ref, agg_ref, wrel_ref, brel_ref,
              wroot_ref, o_ref):
    s = s_ref[...]
    agg = agg_ref[0] + agg_ref[1]
    xs = x_ref[...] * s
    out = (jnp.dot(agg, wrel_ref[...].T, preferred_element_type=jnp.float32,
                   precision=lax.Precision.HIGHEST)
           + brel_ref[...]
           + jnp.dot(xs, wroot_ref[...].T, preferred_element_type=jnp.float32,
                     precision=lax.Precision.HIGHEST))
    out = jnp.where(out >= 0, out, 0.01 * out)
    o_ref[:, :CH] = nx_ref[...]
    o_ref[:, CH:] = jnp.where(s > 0, out, 0.0)


def _finalize(s, x, new_x, agg2, W_rel, b_rel, W_root):
    return pl.pallas_call(
        _out_body,
        grid=(N // BLK,),
        in_specs=[
            pl.BlockSpec((BLK, 1), lambda i: (i, 0)),
            pl.BlockSpec((BLK, CH), lambda i: (i, 0)),
            pl.BlockSpec((BLK, CH), lambda i: (i, 0)),
            pl.BlockSpec((NC, BLK, CH), lambda i: (0, i, 0)),
            pl.BlockSpec((CH, CH), lambda i: (0, 0)),
            pl.BlockSpec((1, CH), lambda i: (0, 0)),
            pl.BlockSpec((CH, CH), lambda i: (0, 0)),
        ],
        out_specs=pl.BlockSpec((BLK, 2 * CH), lambda i: (i, 0)),
        out_shape=jax.ShapeDtypeStruct((N, 2 * CH), jnp.float32),
    )(s, x, new_x, agg2, W_rel, b_rel.reshape(1, CH), W_root)


@jax.jit
def kernel(x, edge_index, new_x, w_attn, W_rel, b_rel, W_root):
    s = _scores(x, w_attn)
    zeros = jnp.zeros((N, CH), jnp.float32)
    agg2 = _sc_call(x, edge_index[0], edge_index[1], s.reshape(N), zeros)
    return _finalize(s, x, new_x, agg2, W_rel, b_rel, W_root)


# speedup vs baseline: 76.5662x; 1.0426x over previous
import functools

import jax
import jax.numpy as jnp
from jax import lax
from jax.experimental import pallas as pl
from jax.experimental.pallas import tpu as pltpu
from jax.experimental.pallas import tpu_sc as plsc

N = 10000
CH = 128
E = 320000
MIN_SCORE = 0.7
TOL = 1e-7

NC = 2
NS = 16
L = 16
NW = NC * NS
EPW = E // NW
VPW = EPW // L
ZR = 16
NCHUNK = N // ZR


def _score_body(x_ref, wa_ref, s_ref):
    x = x_ref[...]
    wa = wa_ref[...]
    score = jnp.sum(x * wa, axis=1, keepdims=True)
    m = jnp.max(score)
    e = jnp.exp(score - m)
    sm = e / jnp.sum(e)
    thr = jnp.minimum(jnp.max(sm) - TOL, MIN_SCORE)
    s_ref[...] = jnp.where(sm > thr, sm, 0.0)


def _scores(x, w_attn):
    return pl.pallas_call(
        _score_body,
        out_shape=jax.ShapeDtypeStruct((N, 1), jnp.float32),
    )(x, w_attn.reshape(1, CH))


UNROLL = 5
RPS = 624


def _sc_body(x_hbm, row_hbm, col_hbm, s_hbm, out_hbm,
             row_v, col_v, s_v, rows_v, zbuf_v, sem, zsem, agg_sh):
    cid = lax.axis_index("c")
    sid = lax.axis_index("s")
    wid = sid * NC + cid
    ebase = wid * EPW

    zero = jnp.zeros((L,), jnp.float32)
    for r in range(ZR):
        for k in range(CH // L):
            zbuf_v[r, pl.ds(k * L, L)] = zero
    rbase = sid * RPS
    zcp = []
    for t in range(RPS // ZR):
        off = rbase + t * ZR
        zcp.append(pltpu.async_copy(
            zbuf_v, agg_sh.at[pl.ds(off, ZR)], zsem))

    @pl.when(sid == NS - 1)
    def _ztail():
        pltpu.async_copy(
            zbuf_v, agg_sh.at[pl.ds(NS * RPS, ZR)], zsem).wait()

    pltpu.sync_copy(row_hbm.at[pl.ds(ebase, EPW)], row_v)
    pltpu.sync_copy(col_hbm.at[pl.ds(ebase, EPW)], col_v)
    pltpu.sync_copy(s_hbm, s_v)
    for c in zcp:
        c.wait()
    plsc.subcore_barrier()

    lanes = lax.iota(jnp.int32, L)

    def _do_chunk(base, w):
        r = row_v[pl.ds(base, L)]
        c = col_v[pl.ds(base, L)]
        pltpu.async_copy(x_hbm.at[r], rows_v, sem).wait()
        for lr in range(L):
            wr = jnp.sum(jnp.where(lanes == lr, w, 0.0))
            for k in range(CH // L):
                sl = pl.ds(k * L, L)
                rows_v[lr, sl] = rows_v[lr, sl] * wr
        pltpu.sync_copy(rows_v, agg_sh.at[c], add=True)

    def step(i, carry):
        base = i * (L * UNROLL)
        ws = [plsc.load_gather(s_v, [row_v[pl.ds(base + u * L, L)]])
              for u in range(UNROLL)]
        tot = ws[0]
        for u in range(1, UNROLL):
            tot = tot + ws[u]

        @pl.when(jnp.sum(tot) > 0.0)
        def _active():
            for u in range(UNROLL):
                @pl.when(jnp.sum(ws[u]) > 0.0)
                def _chunk(u=u):
                    _do_chunk(base + u * L, ws[u])

        return carry

    lax.fori_loop(0, VPW // UNROLL, step, 0)

    plsc.subcore_barrier()

    ocp = []
    for t in range(RPS // ZR):
        off = rbase + t * ZR
        ocp.append(pltpu.async_copy(
            agg_sh.at[pl.ds(off, ZR)], out_hbm.at[cid, pl.ds(off, ZR)], zsem))

    @pl.when(sid == NS - 1)
    def _otail():
        pltpu.async_copy(
            agg_sh.at[pl.ds(NS * RPS, ZR)],
            out_hbm.at[cid, pl.ds(NS * RPS, ZR)], zsem).wait()

    for c in ocp:
        c.wait()


_sc_call = pl.kernel(
    _sc_body,
    out_type=jax.ShapeDtypeStruct((NC, N, CH), jnp.float32),
    mesh=plsc.VectorSubcoreMesh(
        core_axis_name="c", subcore_axis_name="s",
        num_cores=NC, num_subcores=NS),
    scratch_types=[
        pltpu.VMEM((EPW,), jnp.int32),
        pltpu.VMEM((EPW,), jnp.int32),
        pltpu.VMEM((N,), jnp.float32),
        pltpu.VMEM((L, CH), jnp.float32),
        pltpu.VMEM((ZR, CH), jnp.float32),
        pltpu.SemaphoreType.DMA,
        pltpu.SemaphoreType.DMA,
        pltpu.VMEM_SHARED((N, CH), jnp.float32),
    ],
    compiler_params=pltpu.CompilerParams(needs_layout_passes=False),
)


BLK = 1000


def _out_body(s_ref, x_ref, nx_ref, agg_ref, wrel_ref, brel_ref,
              wroot_ref, o_ref):
    s = s_ref[...]
    agg = agg_ref[0] + agg_ref[1]
    xs = x_ref[...] * s
    out = (jnp.dot(agg, wrel_ref[...].T, preferred_element_type=jnp.float32,
                   precision=lax.Precision.HIGHEST)
           + brel_ref[...]
           + jnp.dot(xs, wroot_ref[...].T, preferred_element_type=jnp.float32,
                     precision=lax.Precision.HIGHEST))
    out = jnp.where(out >= 0, out, 0.01 * out)
    o_ref[:, :CH] = nx_ref[...]
    o_ref[:, CH:] = jnp.where(s > 0, out, 0.0)


def _finalize(s, x, new_x, agg2, W_rel, b_rel, W_root):
    return pl.pallas_call(
        _out_body,
        grid=(N // BLK,),
        in_specs=[
            pl.BlockSpec((BLK, 1), lambda i: (i, 0)),
            pl.BlockSpec((BLK, CH), lambda i: (i, 0)),
            pl.BlockSpec((BLK, CH), lambda i: (i, 0)),
            pl.BlockSpec((NC, BLK, CH), lambda i: (0, i, 0)),
            pl.BlockSpec((CH, CH), lambda i: (0, 0)),
            pl.BlockSpec((1, CH), lambda i: (0, 0)),
            pl.BlockSpec((CH, CH), lambda i: (0, 0)),
        ],
        out_specs=pl.BlockSpec((BLK, 2 * CH), lambda i: (i, 0)),
        out_shape=jax.ShapeDtypeStruct((N, 2 * CH), jnp.float32),
    )(s, x, new_x, agg2, W_rel, b_rel.reshape(1, CH), W_root)


@jax.jit
def kernel(x, edge_index, new_x, w_attn, W_rel, b_rel, W_root):
    s = _scores(x, w_attn)
    agg2 = _sc_call(x, edge_index[0], edge_index[1], s.reshape(N))
    return _finalize(s, x, new_x, agg2, W_rel, b_rel, W_root)
